# baseline (speedup 1.0000x reference)
import functools

import jax
import jax.numpy as jnp
from jax import lax
from jax.experimental import pallas as pl
from jax.experimental.pallas import tpu as pltpu
from jax.experimental.pallas import tpu_sc as plsc

N = 10000
D = 128
DH = 64
E = 320000
DG = 16
CH = 128
CHUNKS = 160
E_PAD = 16 * CHUNKS * CH
NP = 10112
RPT = NP // 16

_mesh = plsc.VectorSubcoreMesh(core_axis_name="c", subcore_axis_name="s")
_sc_params = pltpu.CompilerParams(use_tc_tiling_on_sc=False)


def _fill(ref, rows, width, value):
    @pl.loop(0, rows)
    def _(i):
        for j in range(width // 16):
            ref[i, pl.ds(j * 16, 16)] = jnp.full((16,), value, jnp.float32)


ZR = 64


def _zero_shared(zb, acc, r0):
    for k in range(RPT // ZR):
        pltpu.sync_copy(zb, acc.at[pl.ds(r0 + k * ZR, ZR)])
    rem = RPT % ZR
    if rem:
        pltpu.sync_copy(zb.at[pl.ds(0, rem)], acc.at[pl.ds(r0 + (RPT // ZR) * ZR, rem)])


@functools.partial(
    pl.kernel,
    out_type=(jax.ShapeDtypeStruct((2, NP, DH), jnp.float32),
              jax.ShapeDtypeStruct((2, NP, DG), jnp.float32)),
    mesh=_mesh,
    scratch_types=[
        pltpu.VMEM((CHUNKS, CH), jnp.int32),
        pltpu.VMEM((CHUNKS, CH), jnp.int32),
        pltpu.VMEM((CH, DH), jnp.float32),
        pltpu.VMEM((CH, DH), jnp.float32),
        pltpu.VMEM((ZR, DH), jnp.float32),
        pltpu.VMEM((ZR, DG), jnp.float32),
        pltpu.VMEM((CH, DG), jnp.float32),
        pltpu.VMEM_SHARED((NP, DH), jnp.float32),
        pltpu.VMEM_SHARED((NP, DG), jnp.float32),
        pltpu.SemaphoreType.DMA,
        pltpu.SemaphoreType.DMA,
        pltpu.SemaphoreType.DMA,
        pltpu.SemaphoreType.DMA,
        pltpu.SemaphoreType.DMA,
    ],
    compiler_params=_sc_params,
)
def _sc_agg_deg(x_hbm, src_hbm, dst_hbm, out_hbm, deg_hbm,
                sidx, didx, rowsA, rowsB, zb, zbd, ones, acc, dacc,
                semGA, semGB, semSA, semSB, semD):
    cc = lax.axis_index("c")
    ss = lax.axis_index("s")
    wid = cc * 16 + ss
    r0 = ss * RPT
    _fill(zb, ZR, DH, 0.0)
    _fill(zbd, ZR, DG, 0.0)
    _fill(ones, CH, DG, 1.0)
    _zero_shared(zb, acc, r0)
    _zero_shared(zbd, dacc, r0)
    pltpu.sync_copy(src_hbm.at[wid], sidx)
    pltpu.sync_copy(dst_hbm.at[wid], didx)
    plsc.subcore_barrier()

    half = CHUNKS // 2

    def deg_mine(g):
        return lax.select(cc == 0, g < half, g >= half)

    pltpu.async_copy(x_hbm.at[sidx.at[0]], rowsA, semGA)

    @pl.loop(0, CHUNKS, step=2)
    def _(g):
        @pl.when(g > 0)
        def _():
            pltpu.make_async_copy(rowsB, acc.at[didx.at[g - 1]], semSB).wait()
            @pl.when(deg_mine(g - 1))
            def _():
                pltpu.make_async_copy(ones, dacc.at[didx.at[g - 1]], semD).wait()
        pltpu.async_copy(x_hbm.at[sidx.at[g + 1]], rowsB, semGB)

        pltpu.make_async_copy(x_hbm.at[sidx.at[g]], rowsA, semGA).wait()
        pltpu.async_copy(rowsA, acc.at[didx.at[g]], semSA, add=True)
        @pl.when(deg_mine(g))
        def _():
            pltpu.async_copy(ones, dacc.at[didx.at[g]], semD, add=True)

        @pl.when(g + 2 < CHUNKS)
        def _():
            pltpu.make_async_copy(rowsA, acc.at[didx.at[g]], semSA).wait()
            @pl.when(deg_mine(g))
            def _():
                pltpu.make_async_copy(ones, dacc.at[didx.at[g]], semD).wait()
            pltpu.async_copy(x_hbm.at[sidx.at[g + 2]], rowsA, semGA)

        pltpu.make_async_copy(x_hbm.at[sidx.at[g + 1]], rowsB, semGB).wait()
        pltpu.async_copy(rowsB, acc.at[didx.at[g + 1]], semSB, add=True)
        @pl.when(deg_mine(g + 1))
        def _():
            pltpu.async_copy(ones, dacc.at[didx.at[g + 1]], semD, add=True)

    pltpu.make_async_copy(rowsA, acc.at[didx.at[CHUNKS - 2]], semSA).wait()
    pltpu.make_async_copy(rowsB, acc.at[didx.at[CHUNKS - 1]], semSB).wait()
    @pl.when(deg_mine(CHUNKS - 2))
    def _():
        pltpu.make_async_copy(ones, dacc.at[didx.at[CHUNKS - 2]], semD).wait()
    @pl.when(deg_mine(CHUNKS - 1))
    def _():
        pltpu.make_async_copy(ones, dacc.at[didx.at[CHUNKS - 1]], semD).wait()
    plsc.subcore_barrier()
    pltpu.sync_copy(acc.at[pl.ds(r0, RPT)], out_hbm.at[cc, pl.ds(r0, RPT)])
    pltpu.sync_copy(dacc.at[pl.ds(r0, RPT)], deg_hbm.at[cc, pl.ds(r0, RPT)])


@functools.partial(
    pl.kernel,
    out_type=jax.ShapeDtypeStruct((2, NP, DH), jnp.float32),
    mesh=_mesh,
    scratch_types=[
        pltpu.VMEM((CHUNKS // 2, 2 * CH), jnp.int32),
        pltpu.VMEM((CHUNKS // 2, 2 * CH), jnp.int32),
        pltpu.VMEM((2 * CH, DH), jnp.float32),
        pltpu.VMEM((2 * CH, DH), jnp.float32),
        pltpu.VMEM((ZR, DH), jnp.float32),
        pltpu.VMEM_SHARED((NP, DH), jnp.float32),
        pltpu.SemaphoreType.DMA,
        pltpu.SemaphoreType.DMA,
        pltpu.SemaphoreType.DMA,
        pltpu.SemaphoreType.DMA,
    ],
    compiler_params=_sc_params,
)
def _sc_agg(x_hbm, src_hbm, dst_hbm, out_hbm, sidx, didx, rowsA, rowsB, zb,
            acc, semGA, semGB, semSA, semSB):
    PAIRS = CHUNKS // 2
    cc = lax.axis_index("c")
    ss = lax.axis_index("s")
    wid = cc * 16 + ss
    r0 = ss * RPT
    _fill(zb, ZR, DH, 0.0)
    _zero_shared(zb, acc, r0)
    pltpu.sync_copy(src_hbm.at[wid], sidx)
    pltpu.sync_copy(dst_hbm.at[wid], didx)
    plsc.subcore_barrier()

    def gather(p, buf, sem):
        return pltpu.make_async_copy(x_hbm.at[sidx.at[p]], buf, sem)

    def scatter(p, buf, sem):
        return pltpu.make_async_copy(buf, acc.at[didx.at[p]], sem)

    gather(0, rowsA, semGA).start()

    @pl.loop(0, PAIRS, step=2)
    def _(p):
        @pl.when(p > 0)
        def _():
            scatter(p - 1, rowsB, semSB).wait()
        gather(p + 1, rowsB, semGB).start()

        gather(p, rowsA, semGA).wait()
        scatter(p, rowsA, semSA).start(add=True)

        @pl.when(p + 2 < PAIRS)
        def _():
            scatter(p, rowsA, semSA).wait()
            gather(p + 2, rowsA, semGA).start()

        gather(p + 1, rowsB, semGB).wait()
        scatter(p + 1, rowsB, semSB).start(add=True)

    scatter(PAIRS - 2, rowsA, semSA).wait()
    scatter(PAIRS - 1, rowsB, semSB).wait()
    plsc.subcore_barrier()
    pltpu.sync_copy(acc.at[pl.ds(r0, RPT)], out_hbm.at[cc, pl.ds(r0, RPT)])


BR = 1000
_NB = N // BR


def _dot(a, b):
    return jnp.dot(a, b, preferred_element_type=jnp.float32,
                   precision=lax.Precision.HIGHEST)


def _l1_body(x_r, a_r, dg_r, w1s_r, w1n_r, b1_r, w2s_r, hs_r, s2_r):
    deg = dg_r[0, :, 0:1] + dg_r[1, :, 0:1]
    inv = 1.0 / jnp.maximum(deg, 1.0)
    hn = jnp.concatenate([a_r[0], a_r[1]], axis=-1) * inv
    h = _dot(x_r[...], w1s_r[...]) + _dot(hn, w1n_r[...]) + b1_r[...]
    h = jnp.maximum(h, 0.0)
    hs_r[0] = h[:, :DH]
    hs_r[1] = h[:, DH:]
    s2_r[...] = _dot(h, w2s_r[...])


_l1 = pl.pallas_call(
    _l1_body,
    grid=(_NB,),
    in_specs=[
        pl.BlockSpec((BR, D), lambda i: (i, 0)),
        pl.BlockSpec((2, BR, DH), lambda i: (0, i, 0)),
        pl.BlockSpec((2, BR, DG), lambda i: (0, i, 0)),
        pl.BlockSpec((D, D), lambda i: (0, 0)),
        pl.BlockSpec((D, D), lambda i: (0, 0)),
        pl.BlockSpec((1, D), lambda i: (0, 0)),
        pl.BlockSpec((D, D), lambda i: (0, 0)),
    ],
    out_specs=[pl.BlockSpec((2, BR, DH), lambda i: (0, i, 0)),
               pl.BlockSpec((BR, D), lambda i: (i, 0))],
    out_shape=[jax.ShapeDtypeStruct((2, N, DH), jnp.float32),
               jax.ShapeDtypeStruct((N, D), jnp.float32)],
)


def _l2_body(s2_r, a_r, dg_r, w2n_r, b2_r, o_r):
    deg = dg_r[0, :, 0:1] + dg_r[1, :, 0:1]
    inv = 1.0 / jnp.maximum(deg, 1.0)
    hn = jnp.concatenate([a_r[0], a_r[1]], axis=-1) * inv
    o_r[...] = s2_r[...] + _dot(hn, w2n_r[...]) + b2_r[...]


_l2 = pl.pallas_call(
    _l2_body,
    grid=(_NB,),
    in_specs=[
        pl.BlockSpec((BR, D), lambda i: (i, 0)),
        pl.BlockSpec((2, BR, DH), lambda i: (0, i, 0)),
        pl.BlockSpec((2, BR, DG), lambda i: (0, i, 0)),
        pl.BlockSpec((D, D), lambda i: (0, 0)),
        pl.BlockSpec((1, D), lambda i: (0, 0)),
    ],
    out_specs=pl.BlockSpec((BR, D), lambda i: (i, 0)),
    out_shape=jax.ShapeDtypeStruct((N, D), jnp.float32),
)


def kernel(x, edge_index, W1_self, W1_neigh, b1, W2_self, W2_neigh, b2):
    src = edge_index[0].astype(jnp.int32)
    dst = edge_index[1].astype(jnp.int32)
    pad = E_PAD - E
    src_t = jnp.concatenate([src, jnp.zeros((pad,), jnp.int32)]) \
        .reshape(16, CHUNKS, CH)
    dst_t = jnp.concatenate([dst, jnp.full((pad,), N, jnp.int32)]) \
        .reshape(16, CHUNKS, CH)
    src_p = jnp.concatenate([src_t, src_t + N]).reshape(32, CHUNKS, CH)
    dst_p = jnp.concatenate([dst_t, dst_t]).reshape(32, CHUNKS, CH)
    xs = x.reshape(N, 2, DH).swapaxes(0, 1).reshape(2 * N, DH)
    agg1, deg = _sc_agg_deg(xs, src_p, dst_p)
    hs, s2 = _l1(x, agg1, deg, W1_self, W1_neigh, b1.reshape(1, D), W2_self)
    agg2 = _sc_agg(hs.reshape(2 * N, DH),
                   src_p.reshape(32, CHUNKS // 2, 2 * CH),
                   dst_p.reshape(32, CHUNKS // 2, 2 * CH))
    return _l2(s2, agg2, deg, W2_neigh, b2.reshape(1, D))


# speedup vs baseline: 1.3086x; 1.3086x over previous
import functools

import jax
import jax.numpy as jnp
from jax import lax
from jax.experimental import pallas as pl
from jax.experimental.pallas import tpu as pltpu
from jax.experimental.pallas import tpu_sc as plsc

N = 10000
D = 128
DH = 64
E = 320000
DG = 16
CH = 128
CHUNKS = 158
E_PAD = 16 * CHUNKS * CH
NP = 10112
RPT = NP // 16

_mesh = plsc.VectorSubcoreMesh(core_axis_name="c", subcore_axis_name="s")
_sc_params = pltpu.CompilerParams(use_tc_tiling_on_sc=False)


def _fill(ref, rows, width, value):
    @pl.loop(0, rows)
    def _(i):
        for j in range(width // 16):
            ref[i, pl.ds(j * 16, 16)] = jnp.full((16,), value, jnp.float32)


ZR = 64


def _zero_shared(zb, acc, r0):
    for k in range(RPT // ZR):
        pltpu.sync_copy(zb, acc.at[pl.ds(r0 + k * ZR, ZR)])
    rem = RPT % ZR
    if rem:
        pltpu.sync_copy(zb.at[pl.ds(0, rem)], acc.at[pl.ds(r0 + (RPT // ZR) * ZR, rem)])


@functools.partial(
    pl.kernel,
    out_type=(jax.ShapeDtypeStruct((2, NP, DH), jnp.float32),
              jax.ShapeDtypeStruct((2, NP, DG), jnp.float32)),
    mesh=_mesh,
    scratch_types=[
        pltpu.VMEM((CHUNKS, CH), jnp.int32),
        pltpu.VMEM((CHUNKS, CH), jnp.int32),
        pltpu.VMEM((CH, DH), jnp.float32),
        pltpu.VMEM((CH, DH), jnp.float32),
        pltpu.VMEM((ZR, DH), jnp.float32),
        pltpu.VMEM((ZR, DG), jnp.float32),
        pltpu.VMEM((CH, DG), jnp.float32),
        pltpu.VMEM_SHARED((NP, DH), jnp.float32),
        pltpu.VMEM_SHARED((NP, DG), jnp.float32),
        pltpu.SemaphoreType.DMA,
        pltpu.SemaphoreType.DMA,
        pltpu.SemaphoreType.DMA,
        pltpu.SemaphoreType.DMA,
        pltpu.SemaphoreType.DMA,
    ],
    compiler_params=_sc_params,
)
def _sc_agg_deg(x_hbm, src_hbm, dst_hbm, out_hbm, deg_hbm,
                sidx, didx, rowsA, rowsB, zb, zbd, ones, acc, dacc,
                semGA, semGB, semSA, semSB, semD):
    cc = lax.axis_index("c")
    ss = lax.axis_index("s")
    wid = cc * 16 + ss
    r0 = ss * RPT
    _fill(zb, ZR, DH, 0.0)
    _fill(zbd, ZR, DG, 0.0)
    _fill(ones, CH, DG, 1.0)
    _zero_shared(zb, acc, r0)
    _zero_shared(zbd, dacc, r0)
    pltpu.sync_copy(src_hbm.at[wid], sidx)
    pltpu.sync_copy(dst_hbm.at[wid], didx)
    plsc.subcore_barrier()

    half = CHUNKS // 2

    def deg_mine(g):
        return lax.select(cc == 0, g < half, g >= half)

    pltpu.async_copy(x_hbm.at[sidx.at[0]], rowsA, semGA)

    @pl.loop(0, CHUNKS, step=2)
    def _(g):
        @pl.when(g > 0)
        def _():
            pltpu.make_async_copy(rowsB, acc.at[didx.at[g - 1]], semSB).wait()
            @pl.when(deg_mine(g - 1))
            def _():
                pltpu.make_async_copy(ones, dacc.at[didx.at[g - 1]], semD).wait()
        pltpu.async_copy(x_hbm.at[sidx.at[g + 1]], rowsB, semGB)

        pltpu.make_async_copy(x_hbm.at[sidx.at[g]], rowsA, semGA).wait()
        pltpu.async_copy(rowsA, acc.at[didx.at[g]], semSA, add=True)
        @pl.when(deg_mine(g))
        def _():
            pltpu.async_copy(ones, dacc.at[didx.at[g]], semD, add=True)

        @pl.when(g + 2 < CHUNKS)
        def _():
            pltpu.make_async_copy(rowsA, acc.at[didx.at[g]], semSA).wait()
            @pl.when(deg_mine(g))
            def _():
                pltpu.make_async_copy(ones, dacc.at[didx.at[g]], semD).wait()
            pltpu.async_copy(x_hbm.at[sidx.at[g + 2]], rowsA, semGA)

        pltpu.make_async_copy(x_hbm.at[sidx.at[g + 1]], rowsB, semGB).wait()
        pltpu.async_copy(rowsB, acc.at[didx.at[g + 1]], semSB, add=True)
        @pl.when(deg_mine(g + 1))
        def _():
            pltpu.async_copy(ones, dacc.at[didx.at[g + 1]], semD, add=True)

    pltpu.make_async_copy(rowsA, acc.at[didx.at[CHUNKS - 2]], semSA).wait()
    pltpu.make_async_copy(rowsB, acc.at[didx.at[CHUNKS - 1]], semSB).wait()
    @pl.when(deg_mine(CHUNKS - 2))
    def _():
        pltpu.make_async_copy(ones, dacc.at[didx.at[CHUNKS - 2]], semD).wait()
    @pl.when(deg_mine(CHUNKS - 1))
    def _():
        pltpu.make_async_copy(ones, dacc.at[didx.at[CHUNKS - 1]], semD).wait()
    plsc.subcore_barrier()
    pltpu.sync_copy(acc.at[pl.ds(r0, RPT)], out_hbm.at[cc, pl.ds(r0, RPT)])
    pltpu.sync_copy(dacc.at[pl.ds(r0, RPT)], deg_hbm.at[cc, pl.ds(r0, RPT)])


@functools.partial(
    pl.kernel,
    out_type=jax.ShapeDtypeStruct((2, NP, DH), jnp.float32),
    mesh=_mesh,
    scratch_types=[
        pltpu.VMEM((CHUNKS, CH), jnp.int32),
        pltpu.VMEM((CHUNKS, CH), jnp.int32),
        pltpu.VMEM((CH, DH), jnp.float32),
        pltpu.VMEM((CH, DH), jnp.float32),
        pltpu.VMEM((ZR, DH), jnp.float32),
        pltpu.VMEM_SHARED((NP, DH), jnp.float32),
        pltpu.SemaphoreType.DMA,
        pltpu.SemaphoreType.DMA,
        pltpu.SemaphoreType.DMA,
        pltpu.SemaphoreType.DMA,
    ],
    compiler_params=_sc_params,
)
def _sc_agg(x_hbm, src_hbm, dst_hbm, out_hbm, sidx, didx, rowsA, rowsB, zb,
            acc, semGA, semGB, semSA, semSB):
    cc = lax.axis_index("c")
    ss = lax.axis_index("s")
    wid = cc * 16 + ss
    r0 = ss * RPT
    _fill(zb, ZR, DH, 0.0)
    _zero_shared(zb, acc, r0)
    pltpu.sync_copy(src_hbm.at[wid], sidx)
    pltpu.sync_copy(dst_hbm.at[wid], didx)
    plsc.subcore_barrier()

    pltpu.async_copy(x_hbm.at[sidx.at[0]], rowsA, semGA)

    @pl.loop(0, CHUNKS, step=2)
    def _(g):
        @pl.when(g > 0)
        def _():
            pltpu.make_async_copy(rowsB, acc.at[didx.at[g - 1]], semSB).wait()
        pltpu.async_copy(x_hbm.at[sidx.at[g + 1]], rowsB, semGB)

        pltpu.make_async_copy(x_hbm.at[sidx.at[g]], rowsA, semGA).wait()
        pltpu.async_copy(rowsA, acc.at[didx.at[g]], semSA, add=True)

        @pl.when(g + 2 < CHUNKS)
        def _():
            pltpu.make_async_copy(rowsA, acc.at[didx.at[g]], semSA).wait()
            pltpu.async_copy(x_hbm.at[sidx.at[g + 2]], rowsA, semGA)

        pltpu.make_async_copy(x_hbm.at[sidx.at[g + 1]], rowsB, semGB).wait()
        pltpu.async_copy(rowsB, acc.at[didx.at[g + 1]], semSB, add=True)

    pltpu.make_async_copy(rowsA, acc.at[didx.at[CHUNKS - 2]], semSA).wait()
    pltpu.make_async_copy(rowsB, acc.at[didx.at[CHUNKS - 1]], semSB).wait()
    plsc.subcore_barrier()
    pltpu.sync_copy(acc.at[pl.ds(r0, RPT)], out_hbm.at[cc, pl.ds(r0, RPT)])


BR = 1000
_NB = N // BR


def _dot(a, b):
    return jnp.dot(a, b, preferred_element_type=jnp.float32,
                   precision=lax.Precision.HIGHEST)


def _l1_body(x_r, a_r, dg_r, w1s_r, w1n_r, b1_r, w2s_r, hs_r, s2_r):
    deg = dg_r[0, :, 0:1] + dg_r[1, :, 0:1]
    inv = 1.0 / jnp.maximum(deg, 1.0)
    hn = jnp.concatenate([a_r[0], a_r[1]], axis=-1) * inv
    h = _dot(x_r[...], w1s_r[...]) + _dot(hn, w1n_r[...]) + b1_r[...]
    h = jnp.maximum(h, 0.0)
    hs_r[0] = h[:, :DH]
    hs_r[1] = h[:, DH:]
    s2_r[...] = _dot(h, w2s_r[...])


_l1 = pl.pallas_call(
    _l1_body,
    grid=(_NB,),
    in_specs=[
        pl.BlockSpec((BR, D), lambda i: (i, 0)),
        pl.BlockSpec((2, BR, DH), lambda i: (0, i, 0)),
        pl.BlockSpec((2, BR, DG), lambda i: (0, i, 0)),
        pl.BlockSpec((D, D), lambda i: (0, 0)),
        pl.BlockSpec((D, D), lambda i: (0, 0)),
        pl.BlockSpec((1, D), lambda i: (0, 0)),
        pl.BlockSpec((D, D), lambda i: (0, 0)),
    ],
    out_specs=[pl.BlockSpec((2, BR, DH), lambda i: (0, i, 0)),
               pl.BlockSpec((BR, D), lambda i: (i, 0))],
    out_shape=[jax.ShapeDtypeStruct((2, N, DH), jnp.float32),
               jax.ShapeDtypeStruct((N, D), jnp.float32)],
)


def _l2_body(s2_r, a_r, dg_r, w2n_r, b2_r, o_r):
    deg = dg_r[0, :, 0:1] + dg_r[1, :, 0:1]
    inv = 1.0 / jnp.maximum(deg, 1.0)
    hn = jnp.concatenate([a_r[0], a_r[1]], axis=-1) * inv
    o_r[...] = s2_r[...] + _dot(hn, w2n_r[...]) + b2_r[...]


_l2 = pl.pallas_call(
    _l2_body,
    grid=(_NB,),
    in_specs=[
        pl.BlockSpec((BR, D), lambda i: (i, 0)),
        pl.BlockSpec((2, BR, DH), lambda i: (0, i, 0)),
        pl.BlockSpec((2, BR, DG), lambda i: (0, i, 0)),
        pl.BlockSpec((D, D), lambda i: (0, 0)),
        pl.BlockSpec((1, D), lambda i: (0, 0)),
    ],
    out_specs=pl.BlockSpec((BR, D), lambda i: (i, 0)),
    out_shape=jax.ShapeDtypeStruct((N, D), jnp.float32),
)


def kernel(x, edge_index, W1_self, W1_neigh, b1, W2_self, W2_neigh, b2):
    src = edge_index[0].astype(jnp.int32)
    dst = edge_index[1].astype(jnp.int32)
    pad = E_PAD - E
    src_t = jnp.concatenate([src, jnp.zeros((pad,), jnp.int32)]) \
        .reshape(16, CHUNKS, CH)
    dst_t = jnp.concatenate([dst, jnp.full((pad,), N, jnp.int32)]) \
        .reshape(16, CHUNKS, CH)
    src_p = jnp.concatenate([src_t, src_t + N]).reshape(32, CHUNKS, CH)
    dst_p = jnp.concatenate([dst_t, dst_t]).reshape(32, CHUNKS, CH)
    xs = x.reshape(N, 2, DH).swapaxes(0, 1).reshape(2 * N, DH)
    agg1, deg = _sc_agg_deg(xs, src_p, dst_p)
    hs, s2 = _l1(x, agg1, deg, W1_self, W1_neigh, b1.reshape(1, D), W2_self)
    agg2 = _sc_agg(hs.reshape(2 * N, DH), src_p, dst_p)
    return _l2(s2, agg2, deg, W2_neigh, b2.reshape(1, D))


# speedup vs baseline: 1.3250x; 1.0125x over previous
import functools

import jax
import jax.numpy as jnp
from jax import lax
from jax.experimental import pallas as pl
from jax.experimental.pallas import tpu as pltpu
from jax.experimental.pallas import tpu_sc as plsc

N = 10000
D = 128
DH = 64
E = 320000
DG = 16
CH = 128
CHUNKS = 158
E_PAD = 16 * CHUNKS * CH
NP = 10112
RPT = NP // 16

_mesh = plsc.VectorSubcoreMesh(core_axis_name="c", subcore_axis_name="s")
_sc_params = pltpu.CompilerParams(use_tc_tiling_on_sc=False)


def _fill(ref, rows, width, value):
    @pl.loop(0, rows)
    def _(i):
        for j in range(width // 16):
            ref[i, pl.ds(j * 16, 16)] = jnp.full((16,), value, jnp.float32)


ZR = 64


def _zero_shared(zb, acc, r0):
    for k in range(RPT // ZR):
        pltpu.sync_copy(zb, acc.at[pl.ds(r0 + k * ZR, ZR)])
    rem = RPT % ZR
    if rem:
        pltpu.sync_copy(zb.at[pl.ds(0, rem)], acc.at[pl.ds(r0 + (RPT // ZR) * ZR, rem)])


@functools.partial(
    pl.kernel,
    out_type=(jax.ShapeDtypeStruct((2, NP, DH), jnp.float32),
              jax.ShapeDtypeStruct((2, NP, DG), jnp.float32)),
    mesh=_mesh,
    scratch_types=[
        pltpu.VMEM((CHUNKS, CH), jnp.int32),
        pltpu.VMEM((CHUNKS, CH), jnp.int32),
        pltpu.VMEM((CH, DH), jnp.float32),
        pltpu.VMEM((CH, DH), jnp.float32),
        pltpu.VMEM((ZR, DH), jnp.float32),
        pltpu.VMEM((ZR, DG), jnp.float32),
        pltpu.VMEM((CH, DG), jnp.float32),
        pltpu.VMEM_SHARED((NP, DH), jnp.float32),
        pltpu.VMEM_SHARED((NP, DG), jnp.float32),
        pltpu.SemaphoreType.DMA,
        pltpu.SemaphoreType.DMA,
        pltpu.SemaphoreType.DMA,
        pltpu.SemaphoreType.DMA,
        pltpu.SemaphoreType.DMA,
    ],
    compiler_params=_sc_params,
)
def _sc_agg_deg(x_hbm, src_hbm, dst_hbm, out_hbm, deg_hbm,
                sidx, didx, rowsA, rowsB, zb, zbd, ones, acc, dacc,
                semGA, semGB, semSA, semSB, semD):
    cc = lax.axis_index("c")
    ss = lax.axis_index("s")
    wid = cc * 16 + ss
    r0 = ss * RPT
    _fill(zb, ZR, DH, 0.0)
    _fill(zbd, ZR, DG, 0.0)
    _fill(ones, CH, DG, 1.0)
    _zero_shared(zb, acc, r0)
    _zero_shared(zbd, dacc, r0)
    pltpu.sync_copy(src_hbm.at[wid], sidx)
    pltpu.sync_copy(dst_hbm.at[wid], didx)
    plsc.subcore_barrier()

    half = CHUNKS // 2

    def deg_mine(g):
        return lax.select(cc == 0, g < half, g >= half)

    pltpu.async_copy(x_hbm.at[sidx.at[0]], rowsA, semGA)

    @pl.loop(0, CHUNKS, step=2)
    def _(g):
        @pl.when(g > 0)
        def _():
            pltpu.make_async_copy(rowsB, acc.at[didx.at[g - 1]], semSB).wait()
            @pl.when(deg_mine(g - 1))
            def _():
                pltpu.make_async_copy(ones, dacc.at[didx.at[g - 1]], semD).wait()
        pltpu.async_copy(x_hbm.at[sidx.at[g + 1]], rowsB, semGB)

        pltpu.make_async_copy(x_hbm.at[sidx.at[g]], rowsA, semGA).wait()
        pltpu.async_copy(rowsA, acc.at[didx.at[g]], semSA, add=True)
        @pl.when(deg_mine(g))
        def _():
            pltpu.async_copy(ones, dacc.at[didx.at[g]], semD, add=True)

        @pl.when(g + 2 < CHUNKS)
        def _():
            pltpu.make_async_copy(rowsA, acc.at[didx.at[g]], semSA).wait()
            @pl.when(deg_mine(g))
            def _():
                pltpu.make_async_copy(ones, dacc.at[didx.at[g]], semD).wait()
            pltpu.async_copy(x_hbm.at[sidx.at[g + 2]], rowsA, semGA)

        pltpu.make_async_copy(x_hbm.at[sidx.at[g + 1]], rowsB, semGB).wait()
        pltpu.async_copy(rowsB, acc.at[didx.at[g + 1]], semSB, add=True)
        @pl.when(deg_mine(g + 1))
        def _():
            pltpu.async_copy(ones, dacc.at[didx.at[g + 1]], semD, add=True)

    pltpu.make_async_copy(rowsA, acc.at[didx.at[CHUNKS - 2]], semSA).wait()
    pltpu.make_async_copy(rowsB, acc.at[didx.at[CHUNKS - 1]], semSB).wait()
    @pl.when(deg_mine(CHUNKS - 2))
    def _():
        pltpu.make_async_copy(ones, dacc.at[didx.at[CHUNKS - 2]], semD).wait()
    @pl.when(deg_mine(CHUNKS - 1))
    def _():
        pltpu.make_async_copy(ones, dacc.at[didx.at[CHUNKS - 1]], semD).wait()
    plsc.subcore_barrier()
    pltpu.sync_copy(acc.at[pl.ds(r0, RPT)], out_hbm.at[cc, pl.ds(r0, RPT)])
    pltpu.sync_copy(dacc.at[pl.ds(r0, RPT)], deg_hbm.at[cc, pl.ds(r0, RPT)])


@functools.partial(
    pl.kernel,
    out_type=jax.ShapeDtypeStruct((2, NP, DH), jnp.float32),
    mesh=_mesh,
    scratch_types=[
        pltpu.VMEM((CHUNKS, CH), jnp.int32),
        pltpu.VMEM((CHUNKS, CH), jnp.int32),
        pltpu.VMEM((CH, DH), jnp.float32),
        pltpu.VMEM((CH, DH), jnp.float32),
        pltpu.VMEM((ZR, DH), jnp.float32),
        pltpu.VMEM_SHARED((NP, DH), jnp.float32),
        pltpu.SemaphoreType.DMA,
        pltpu.SemaphoreType.DMA,
        pltpu.SemaphoreType.DMA,
        pltpu.SemaphoreType.DMA,
    ],
    compiler_params=_sc_params,
)
def _sc_agg(x_hbm, src_hbm, dst_hbm, out_hbm, sidx, didx, rowsA, rowsB, zb,
            acc, semGA, semGB, semSA, semSB):
    cc = lax.axis_index("c")
    ss = lax.axis_index("s")
    wid = cc * 16 + ss
    r0 = ss * RPT
    _fill(zb, ZR, DH, 0.0)
    _zero_shared(zb, acc, r0)
    pltpu.sync_copy(src_hbm.at[wid], sidx)
    pltpu.sync_copy(dst_hbm.at[wid], didx)
    plsc.subcore_barrier()

    pltpu.async_copy(x_hbm.at[sidx.at[0]], rowsA, semGA)

    @pl.loop(0, CHUNKS, step=2)
    def _(g):
        @pl.when(g > 0)
        def _():
            pltpu.make_async_copy(rowsB, acc.at[didx.at[g - 1]], semSB).wait()
        pltpu.async_copy(x_hbm.at[sidx.at[g + 1]], rowsB, semGB)

        pltpu.make_async_copy(x_hbm.at[sidx.at[g]], rowsA, semGA).wait()
        pltpu.async_copy(rowsA, acc.at[didx.at[g]], semSA, add=True)

        @pl.when(g + 2 < CHUNKS)
        def _():
            pltpu.make_async_copy(rowsA, acc.at[didx.at[g]], semSA).wait()
            pltpu.async_copy(x_hbm.at[sidx.at[g + 2]], rowsA, semGA)

        pltpu.make_async_copy(x_hbm.at[sidx.at[g + 1]], rowsB, semGB).wait()
        pltpu.async_copy(rowsB, acc.at[didx.at[g + 1]], semSB, add=True)

    pltpu.make_async_copy(rowsA, acc.at[didx.at[CHUNKS - 2]], semSA).wait()
    pltpu.make_async_copy(rowsB, acc.at[didx.at[CHUNKS - 1]], semSB).wait()
    plsc.subcore_barrier()
    pltpu.sync_copy(acc.at[pl.ds(r0, RPT)], out_hbm.at[cc, pl.ds(r0, RPT)])


BR = 1000
_NB = N // BR


def _dot(a, b):
    return jnp.dot(a, b, preferred_element_type=jnp.float32,
                   precision=lax.Precision.HIGHEST)


def _pre_body(x_r, w1s_r, b1_r, s1_r):
    s1_r[...] = _dot(x_r[...], w1s_r[...]) + b1_r[...]


_pre = pl.pallas_call(
    _pre_body,
    grid=(_NB,),
    in_specs=[
        pl.BlockSpec((BR, D), lambda i: (i, 0)),
        pl.BlockSpec((D, D), lambda i: (0, 0)),
        pl.BlockSpec((1, D), lambda i: (0, 0)),
    ],
    out_specs=pl.BlockSpec((BR, D), lambda i: (i, 0)),
    out_shape=jax.ShapeDtypeStruct((N, D), jnp.float32),
)


def _l1_body(s1_r, a_r, dg_r, w1n_r, w2s_r, hs_r, s2_r):
    deg = dg_r[0, :, 0:1] + dg_r[1, :, 0:1]
    inv = 1.0 / jnp.maximum(deg, 1.0)
    hn = jnp.concatenate([a_r[0], a_r[1]], axis=-1) * inv
    h = jnp.maximum(s1_r[...] + _dot(hn, w1n_r[...]), 0.0)
    hs_r[0] = h[:, :DH]
    hs_r[1] = h[:, DH:]
    s2_r[...] = _dot(h, w2s_r[...])


_l1 = pl.pallas_call(
    _l1_body,
    grid=(_NB,),
    in_specs=[
        pl.BlockSpec((BR, D), lambda i: (i, 0)),
        pl.BlockSpec((2, BR, DH), lambda i: (0, i, 0)),
        pl.BlockSpec((2, BR, DG), lambda i: (0, i, 0)),
        pl.BlockSpec((D, D), lambda i: (0, 0)),
        pl.BlockSpec((D, D), lambda i: (0, 0)),
    ],
    out_specs=[pl.BlockSpec((2, BR, DH), lambda i: (0, i, 0)),
               pl.BlockSpec((BR, D), lambda i: (i, 0))],
    out_shape=[jax.ShapeDtypeStruct((2, N, DH), jnp.float32),
               jax.ShapeDtypeStruct((N, D), jnp.float32)],
)


def _l2_body(s2_r, a_r, dg_r, w2n_r, b2_r, o_r):
    deg = dg_r[0, :, 0:1] + dg_r[1, :, 0:1]
    inv = 1.0 / jnp.maximum(deg, 1.0)
    hn = jnp.concatenate([a_r[0], a_r[1]], axis=-1) * inv
    o_r[...] = s2_r[...] + _dot(hn, w2n_r[...]) + b2_r[...]


_l2 = pl.pallas_call(
    _l2_body,
    grid=(_NB,),
    in_specs=[
        pl.BlockSpec((BR, D), lambda i: (i, 0)),
        pl.BlockSpec((2, BR, DH), lambda i: (0, i, 0)),
        pl.BlockSpec((2, BR, DG), lambda i: (0, i, 0)),
        pl.BlockSpec((D, D), lambda i: (0, 0)),
        pl.BlockSpec((1, D), lambda i: (0, 0)),
    ],
    out_specs=pl.BlockSpec((BR, D), lambda i: (i, 0)),
    out_shape=jax.ShapeDtypeStruct((N, D), jnp.float32),
)


def kernel(x, edge_index, W1_self, W1_neigh, b1, W2_self, W2_neigh, b2):
    src = edge_index[0].astype(jnp.int32)
    dst = edge_index[1].astype(jnp.int32)
    pad = E_PAD - E
    src_t = jnp.concatenate([src, jnp.zeros((pad,), jnp.int32)]) \
        .reshape(16, CHUNKS, CH)
    dst_t = jnp.concatenate([dst, jnp.full((pad,), N, jnp.int32)]) \
        .reshape(16, CHUNKS, CH)
    src_p = jnp.concatenate([src_t, src_t + N]).reshape(32, CHUNKS, CH)
    dst_p = jnp.concatenate([dst_t, dst_t]).reshape(32, CHUNKS, CH)
    xs = x.reshape(N, 2, DH).swapaxes(0, 1).reshape(2 * N, DH)
    agg1, deg = _sc_agg_deg(xs, src_p, dst_p)
    s1 = _pre(x, W1_self, b1.reshape(1, D))
    hs, s2 = _l1(s1, agg1, deg, W1_neigh, W2_self)
    agg2 = _sc_agg(hs.reshape(2 * N, DH), src_p, dst_p)
    return _l2(s2, agg2, deg, W2_neigh, b2.reshape(1, D))


# speedup vs baseline: 1.3919x; 1.0505x over previous
import functools

import jax
import jax.numpy as jnp
from jax import lax
from jax.experimental import pallas as pl
from jax.experimental.pallas import tpu as pltpu
from jax.experimental.pallas import tpu_sc as plsc

N = 10000
D = 128
DH = 64
E = 320000
DG = 16
CH = 128
CHUNKS = 158
E_PAD = 16 * CHUNKS * CH
NP = 10112
RPT = NP // 16

_mesh = plsc.VectorSubcoreMesh(core_axis_name="c", subcore_axis_name="s")
_sc_params = pltpu.CompilerParams(use_tc_tiling_on_sc=False)


def _fill(ref, rows, width, value):
    @pl.loop(0, rows)
    def _(i):
        for j in range(width // 16):
            ref[i, pl.ds(j * 16, 16)] = jnp.full((16,), value, jnp.float32)


ZR = 64


def _zero_shared(zb, acc, r0):
    for k in range(RPT // ZR):
        pltpu.sync_copy(zb, acc.at[pl.ds(r0 + k * ZR, ZR)])
    rem = RPT % ZR
    if rem:
        pltpu.sync_copy(zb.at[pl.ds(0, rem)], acc.at[pl.ds(r0 + (RPT // ZR) * ZR, rem)])


@functools.partial(
    pl.kernel,
    out_type=(jax.ShapeDtypeStruct((2, NP, DH), jnp.float32),
              jax.ShapeDtypeStruct((2, NP, DG), jnp.float32)),
    mesh=_mesh,
    scratch_types=[
        pltpu.VMEM((CHUNKS, CH), jnp.int32),
        pltpu.VMEM((CHUNKS, CH), jnp.int32),
        pltpu.VMEM((CH, DH), jnp.float32),
        pltpu.VMEM((CH, DH), jnp.float32),
        pltpu.VMEM((ZR, DH), jnp.float32),
        pltpu.VMEM((ZR, DG), jnp.float32),
        pltpu.VMEM((CH, DG), jnp.float32),
        pltpu.VMEM_SHARED((NP, DH), jnp.float32),
        pltpu.VMEM_SHARED((NP, DG), jnp.float32),
        pltpu.SemaphoreType.DMA,
        pltpu.SemaphoreType.DMA,
        pltpu.SemaphoreType.DMA,
        pltpu.SemaphoreType.DMA,
        pltpu.SemaphoreType.DMA,
    ],
    compiler_params=_sc_params,
)
def _sc_agg_deg(x_hbm, src_hbm, dst_hbm, out_hbm, deg_hbm,
                sidx, didx, rowsA, rowsB, zb, zbd, ones, acc, dacc,
                semGA, semGB, semSA, semSB, semD):
    cc = lax.axis_index("c")
    ss = lax.axis_index("s")
    wid = cc * 16 + ss
    r0 = ss * RPT
    _fill(zb, ZR, DH, 0.0)
    _fill(zbd, ZR, DG, 0.0)
    _fill(ones, CH, DG, 1.0)
    _zero_shared(zb, acc, r0)
    _zero_shared(zbd, dacc, r0)
    pltpu.sync_copy(src_hbm.at[wid], sidx)
    pltpu.sync_copy(dst_hbm.at[wid], didx)
    plsc.subcore_barrier()

    half = CHUNKS // 2

    def deg_mine(g):
        return lax.select(cc == 0, g < half, g >= half)

    pltpu.async_copy(x_hbm.at[sidx.at[0]], rowsA, semGA)

    @pl.loop(0, CHUNKS, step=2)
    def _(g):
        @pl.when(g > 0)
        def _():
            pltpu.make_async_copy(rowsB, acc.at[didx.at[g - 1]], semSB).wait()
            @pl.when(deg_mine(g - 1))
            def _():
                pltpu.make_async_copy(ones, dacc.at[didx.at[g - 1]], semD).wait()
        pltpu.async_copy(x_hbm.at[sidx.at[g + 1]], rowsB, semGB)

        pltpu.make_async_copy(x_hbm.at[sidx.at[g]], rowsA, semGA).wait()
        pltpu.async_copy(rowsA, acc.at[didx.at[g]], semSA, add=True)
        @pl.when(deg_mine(g))
        def _():
            pltpu.async_copy(ones, dacc.at[didx.at[g]], semD, add=True)

        @pl.when(g + 2 < CHUNKS)
        def _():
            pltpu.make_async_copy(rowsA, acc.at[didx.at[g]], semSA).wait()
            @pl.when(deg_mine(g))
            def _():
                pltpu.make_async_copy(ones, dacc.at[didx.at[g]], semD).wait()
            pltpu.async_copy(x_hbm.at[sidx.at[g + 2]], rowsA, semGA)

        pltpu.make_async_copy(x_hbm.at[sidx.at[g + 1]], rowsB, semGB).wait()
        pltpu.async_copy(rowsB, acc.at[didx.at[g + 1]], semSB, add=True)
        @pl.when(deg_mine(g + 1))
        def _():
            pltpu.async_copy(ones, dacc.at[didx.at[g + 1]], semD, add=True)

    pltpu.make_async_copy(rowsA, acc.at[didx.at[CHUNKS - 2]], semSA).wait()
    pltpu.make_async_copy(rowsB, acc.at[didx.at[CHUNKS - 1]], semSB).wait()
    @pl.when(deg_mine(CHUNKS - 2))
    def _():
        pltpu.make_async_copy(ones, dacc.at[didx.at[CHUNKS - 2]], semD).wait()
    @pl.when(deg_mine(CHUNKS - 1))
    def _():
        pltpu.make_async_copy(ones, dacc.at[didx.at[CHUNKS - 1]], semD).wait()
    plsc.subcore_barrier()
    pltpu.sync_copy(acc.at[pl.ds(r0, RPT)], out_hbm.at[cc, pl.ds(r0, RPT)])
    pltpu.sync_copy(dacc.at[pl.ds(r0, RPT)], deg_hbm.at[cc, pl.ds(r0, RPT)])


@functools.partial(
    pl.kernel,
    out_type=jax.ShapeDtypeStruct((2, NP, DH), jnp.float32),
    mesh=_mesh,
    scratch_types=[
        pltpu.VMEM((CHUNKS, CH), jnp.int32),
        pltpu.VMEM((CHUNKS, CH), jnp.int32),
        pltpu.VMEM((CH, DH), jnp.float32),
        pltpu.VMEM((CH, DH), jnp.float32),
        pltpu.VMEM((ZR, DH), jnp.float32),
        pltpu.VMEM_SHARED((NP, DH), jnp.float32),
        pltpu.SemaphoreType.DMA,
        pltpu.SemaphoreType.DMA,
        pltpu.SemaphoreType.DMA,
        pltpu.SemaphoreType.DMA,
    ],
    compiler_params=_sc_params,
)
def _sc_agg(x_hbm, src_hbm, dst_hbm, out_hbm, sidx, didx, rowsA, rowsB, zb,
            acc, semGA, semGB, semSA, semSB):
    cc = lax.axis_index("c")
    ss = lax.axis_index("s")
    wid = cc * 16 + ss
    r0 = ss * RPT
    _fill(zb, ZR, DH, 0.0)
    _zero_shared(zb, acc, r0)
    pltpu.sync_copy(src_hbm.at[wid], sidx)
    pltpu.sync_copy(dst_hbm.at[wid], didx)
    plsc.subcore_barrier()

    pltpu.async_copy(x_hbm.at[sidx.at[0]], rowsA, semGA)

    @pl.loop(0, CHUNKS, step=2)
    def _(g):
        @pl.when(g > 0)
        def _():
            pltpu.make_async_copy(rowsB, acc.at[didx.at[g - 1]], semSB).wait()
        pltpu.async_copy(x_hbm.at[sidx.at[g + 1]], rowsB, semGB)

        pltpu.make_async_copy(x_hbm.at[sidx.at[g]], rowsA, semGA).wait()
        pltpu.async_copy(rowsA, acc.at[didx.at[g]], semSA, add=True)

        @pl.when(g + 2 < CHUNKS)
        def _():
            pltpu.make_async_copy(rowsA, acc.at[didx.at[g]], semSA).wait()
            pltpu.async_copy(x_hbm.at[sidx.at[g + 2]], rowsA, semGA)

        pltpu.make_async_copy(x_hbm.at[sidx.at[g + 1]], rowsB, semGB).wait()
        pltpu.async_copy(rowsB, acc.at[didx.at[g + 1]], semSB, add=True)

    pltpu.make_async_copy(rowsA, acc.at[didx.at[CHUNKS - 2]], semSA).wait()
    pltpu.make_async_copy(rowsB, acc.at[didx.at[CHUNKS - 1]], semSB).wait()
    plsc.subcore_barrier()
    pltpu.sync_copy(acc.at[pl.ds(r0, RPT)], out_hbm.at[cc, pl.ds(r0, RPT)])


BR = 2000
_NB = N // BR


def _dot(a, b):
    return jnp.dot(a, b, preferred_element_type=jnp.float32)


def _pre_body(x_r, w1s_r, b1_r, s1_r):
    s1_r[...] = _dot(x_r[...], w1s_r[...]) + b1_r[...]


_pre = pl.pallas_call(
    _pre_body,
    grid=(_NB,),
    in_specs=[
        pl.BlockSpec((BR, D), lambda i: (i, 0)),
        pl.BlockSpec((D, D), lambda i: (0, 0)),
        pl.BlockSpec((1, D), lambda i: (0, 0)),
    ],
    out_specs=pl.BlockSpec((BR, D), lambda i: (i, 0)),
    out_shape=jax.ShapeDtypeStruct((N, D), jnp.float32),
)


def _l1_body(s1_r, a_r, dg_r, w1n_r, w2s_r, hs_r, s2_r):
    deg = dg_r[0, :, 0:1] + dg_r[1, :, 0:1]
    inv = 1.0 / jnp.maximum(deg, 1.0)
    hn = jnp.concatenate([a_r[0], a_r[1]], axis=-1) * inv
    h = jnp.maximum(s1_r[...] + _dot(hn, w1n_r[...]), 0.0)
    hs_r[0] = h[:, :DH]
    hs_r[1] = h[:, DH:]
    s2_r[...] = _dot(h, w2s_r[...])


_l1 = pl.pallas_call(
    _l1_body,
    grid=(_NB,),
    in_specs=[
        pl.BlockSpec((BR, D), lambda i: (i, 0)),
        pl.BlockSpec((2, BR, DH), lambda i: (0, i, 0)),
        pl.BlockSpec((2, BR, DG), lambda i: (0, i, 0)),
        pl.BlockSpec((D, D), lambda i: (0, 0)),
        pl.BlockSpec((D, D), lambda i: (0, 0)),
    ],
    out_specs=[pl.BlockSpec((2, BR, DH), lambda i: (0, i, 0)),
               pl.BlockSpec((BR, D), lambda i: (i, 0))],
    out_shape=[jax.ShapeDtypeStruct((2, N, DH), jnp.float32),
               jax.ShapeDtypeStruct((N, D), jnp.float32)],
)


def _l2_body(s2_r, a_r, dg_r, w2n_r, b2_r, o_r):
    deg = dg_r[0, :, 0:1] + dg_r[1, :, 0:1]
    inv = 1.0 / jnp.maximum(deg, 1.0)
    hn = jnp.concatenate([a_r[0], a_r[1]], axis=-1) * inv
    o_r[...] = s2_r[...] + _dot(hn, w2n_r[...]) + b2_r[...]


_l2 = pl.pallas_call(
    _l2_body,
    grid=(_NB,),
    in_specs=[
        pl.BlockSpec((BR, D), lambda i: (i, 0)),
        pl.BlockSpec((2, BR, DH), lambda i: (0, i, 0)),
        pl.BlockSpec((2, BR, DG), lambda i: (0, i, 0)),
        pl.BlockSpec((D, D), lambda i: (0, 0)),
        pl.BlockSpec((1, D), lambda i: (0, 0)),
    ],
    out_specs=pl.BlockSpec((BR, D), lambda i: (i, 0)),
    out_shape=jax.ShapeDtypeStruct((N, D), jnp.float32),
)


def kernel(x, edge_index, W1_self, W1_neigh, b1, W2_self, W2_neigh, b2):
    src = edge_index[0].astype(jnp.int32)
    dst = edge_index[1].astype(jnp.int32)
    pad = E_PAD - E
    src_t = jnp.concatenate([src, jnp.zeros((pad,), jnp.int32)]) \
        .reshape(16, CHUNKS, CH)
    dst_t = jnp.concatenate([dst, jnp.full((pad,), N, jnp.int32)]) \
        .reshape(16, CHUNKS, CH)
    src_p = jnp.concatenate([src_t, src_t + N]).reshape(32, CHUNKS, CH)
    dst_p = jnp.concatenate([dst_t, dst_t]).reshape(32, CHUNKS, CH)
    xs = x.reshape(N, 2, DH).swapaxes(0, 1).reshape(2 * N, DH)
    agg1, deg = _sc_agg_deg(xs, src_p, dst_p)
    s1 = _pre(x, W1_self, b1.reshape(1, D))
    hs, s2 = _l1(s1, agg1, deg, W1_neigh, W2_self)
    agg2 = _sc_agg(hs.reshape(2 * N, DH), src_p, dst_p)
    return _l2(s2, agg2, deg, W2_neigh, b2.reshape(1, D))
